# baseline (speedup 1.0000x reference)
import jax
import jax.numpy as jnp
from jax import lax
from jax.experimental import pallas as pl
from jax.experimental.pallas import tpu as pltpu
from jax.experimental.pallas import tpu_sc as plsc

BATCH = 4096
SEQ = 200
DIM = 64
LANES = 16

NC = 2
NS = 16
NW = NC * NS
SUB = 64
ROWS_PER_W = BATCH * SEQ // NW
NSUB = ROWS_PER_W // SUB
NBUF = 4
PSP = SUB * LANES


def _body(idx_hbm, table_hbm, pos2_hbm, par_hbm, out_hbm, *scratch):
    idx_v = scratch[0]
    pos2_v = scratch[1]
    rin = scratch[2:2 + NBUF]
    rout = scratch[2 + NBUF:2 + 2 * NBUF]
    psp = scratch[2 + 2 * NBUF:2 + 3 * NBUF]
    gsem = scratch[2 + 3 * NBUF:2 + 4 * NBUF]
    ssem = scratch[2 + 4 * NBUF:2 + 5 * NBUF]
    psem = scratch[2 + 5 * NBUF:2 + 6 * NBUF]

    c = lax.axis_index("c")
    s = lax.axis_index("s")
    wid = s * NC + c

    pltpu.sync_copy(idx_hbm.at[wid], idx_v)
    pltpu.sync_copy(pos2_hbm, pos2_v)

    def idx_ref(g_half, parity):
        return idx_v.at[g_half, pl.ds(parity * SUB, SUB)]

    for b in range(NBUF):
        pltpu.make_async_copy(
            table_hbm.at[idx_ref(b // 2, b % 2)], rin[b], gsem[b]
        ).start()
        pltpu.make_async_copy(par_hbm.at[wid, b], psp[b], psem[b]).start()

    @pl.loop(0, NSUB, step=NBUF)
    def _round(g0):
        for b in range(NBUF):
            g = g0 + b
            pltpu.make_async_copy(
                table_hbm.at[idx_ref(g0, 0)], rin[b], gsem[b]
            ).wait()
            pltpu.make_async_copy(par_hbm.at[wid, g], psp[b], psem[b]).wait()

            @pl.when(g0 > 0)
            def _():
                pltpu.make_async_copy(
                    rout[b], out_hbm.at[g], ssem[b]
                ).wait()

            r0d = lax.rem(g * (SUB // 2), SEQ // 2)

            @plsc.parallel_loop(0, SUB // 2, unroll=4)
            def _add(jj):
                for h in range(2):
                    pv = psp[b][pl.ds((2 * jj + h) * LANES, LANES)]
                    m = pv != 0
                    for k in range(DIM // LANES):
                        lo = rin[b][2 * jj + h, pl.ds(k * LANES, LANES)]
                        hi = rin[b][2 * jj + h, pl.ds(DIM + k * LANES, LANES)]
                        p = pos2_v[r0d + jj, pl.ds(h * DIM + k * LANES, LANES)]
                        rout[b][jj, pl.ds(h * DIM + k * LANES, LANES)] = (
                            jnp.where(m, hi, lo) + p
                        )

            pltpu.make_async_copy(
                rout[b], out_hbm.at[wid * NSUB + g], ssem[b]
            ).start()

            nxt = g + NBUF

            @pl.when(nxt < NSUB)
            def _():
                pltpu.make_async_copy(
                    table_hbm.at[idx_ref(g0 // 2 + (b + NBUF) // 2, b % 2)],
                    rin[b],
                    gsem[b],
                ).start()
                pltpu.make_async_copy(
                    par_hbm.at[wid, nxt], psp[b], psem[b]
                ).start()

    for b in range(NBUF):
        pltpu.make_async_copy(
            rout[b], out_hbm.at[NSUB - NBUF + b], ssem[b]
        ).wait()


_scratch = (
    [
        pltpu.VMEM((NSUB // 2, 2 * SUB), jnp.int32),
        pltpu.VMEM((SEQ, 2 * DIM), jnp.float32),
    ]
    + [pltpu.VMEM((SUB, 2 * DIM), jnp.float32) for _ in range(NBUF)]
    + [pltpu.VMEM((SUB // 2, 2 * DIM), jnp.float32) for _ in range(NBUF)]
    + [pltpu.VMEM((PSP,), jnp.int32) for _ in range(NBUF)]
    + [pltpu.SemaphoreType.DMA for _ in range(3 * NBUF)]
)

_kern = pl.kernel(
    _body,
    out_type=jax.ShapeDtypeStruct((NW * NSUB, SUB // 2, 2 * DIM), jnp.float32),
    mesh=plsc.VectorSubcoreMesh(core_axis_name="c", subcore_axis_name="s"),
    scratch_types=_scratch,
    compiler_params=pltpu.CompilerParams(use_tc_tiling_on_sc=True),
    name="token_pos_embed_sc",
)


@jax.jit
def kernel(inputs, token_table, pos_table):
    b, l = inputs.shape
    _, d = token_table.shape
    flat = inputs.astype(jnp.int32).reshape(NW, ROWS_PER_W)
    idx2 = (flat >> 1).reshape(NW, NSUB // 2, 2 * SUB)
    par = jnp.broadcast_to(
        (flat & 1)[:, :, None], (NW, ROWS_PER_W, LANES)
    ).reshape(NW, NSUB, PSP)
    pos2 = jnp.concatenate([pos_table, pos_table], axis=0).reshape(
        SEQ, 2 * DIM
    )
    out = _kern(idx2, token_table.reshape(-1, 2 * DIM), pos2, par)
    return out.reshape(b, l, d)


# speedup vs baseline: 1.1524x; 1.1524x over previous
import jax
import jax.numpy as jnp
from jax import lax
from jax.experimental import pallas as pl
from jax.experimental.pallas import tpu as pltpu
from jax.experimental.pallas import tpu_sc as plsc

BATCH = 4096
SEQ = 200
DIM = 64
LANES = 16

NC = 2
NS = 16
NW = NC * NS
SUB = 64
ROWS_PER_W = BATCH * SEQ // NW
NSUB = ROWS_PER_W // SUB
NBUF = 4
IDXROWS = NSUB // 2
IDXVECS = IDXROWS * (2 * SUB // LANES)


def _body(ridx_hbm, table_hbm, pos2_hbm, out_hbm, *scratch):
    idx_v = scratch[0]
    ridx_v = scratch[1]
    pos2_v = scratch[2]
    rin = scratch[3:3 + NBUF]
    rout = scratch[3 + NBUF:3 + 2 * NBUF]
    gsem = scratch[3 + 2 * NBUF:3 + 3 * NBUF]
    ssem = scratch[3 + 3 * NBUF:3 + 4 * NBUF]

    c = lax.axis_index("c")
    s = lax.axis_index("s")
    wid = s * NC + c

    pltpu.sync_copy(ridx_hbm.at[wid], ridx_v)
    pltpu.sync_copy(pos2_hbm, pos2_v)

    @plsc.parallel_loop(0, IDXVECS, unroll=8)
    def _halve(t):
        r = t // (2 * SUB // LANES)
        col = (t % (2 * SUB // LANES)) * LANES
        idx_v[r, pl.ds(col, LANES)] = ridx_v[r, pl.ds(col, LANES)] >> 1

    def idx_ref(g_half, parity):
        return idx_v.at[g_half, pl.ds(parity * SUB, SUB)]

    for b in range(NBUF):
        pltpu.make_async_copy(
            table_hbm.at[idx_ref(b // 2, b % 2)], rin[b], gsem[b]
        ).start()

    @pl.loop(0, NSUB, step=NBUF)
    def _round(g0):
        for b in range(NBUF):
            g = g0 + b
            gh = g // 2
            off = lax.rem(g, 2) * SUB
            pltpu.make_async_copy(
                table_hbm.at[idx_ref(g0, 0)], rin[b], gsem[b]
            ).wait()

            @pl.when(g0 > 0)
            def _():
                pltpu.make_async_copy(
                    rout[b], out_hbm.at[g], ssem[b]
                ).wait()

            r0d = lax.rem(g * (SUB // 2), SEQ // 2)

            @plsc.parallel_loop(0, SUB // 2, unroll=4)
            def _add(jj):
                grp = off + (jj // 8) * LANES
                pvec = (ridx_v[gh, pl.ds(grp, LANES)] & 1).astype(jnp.float32)
                lane0 = lax.rem(jj, 8) * 2
                for h in range(2):
                    lanes = jnp.full((LANES,), lane0 + h, jnp.int32)
                    ps = pvec.at[lanes].get(mode="promise_in_bounds")
                    row = 2 * jj + h
                    for k in range(DIM // LANES):
                        lo = rin[b][row, pl.ds(k * LANES, LANES)]
                        hi = rin[b][row, pl.ds(DIM + k * LANES, LANES)]
                        p = pos2_v[r0d + jj, pl.ds(h * DIM + k * LANES, LANES)]
                        rout[b][jj, pl.ds(h * DIM + k * LANES, LANES)] = (
                            lo + ps * (hi - lo) + p
                        )

            pltpu.make_async_copy(
                rout[b], out_hbm.at[wid * NSUB + g], ssem[b]
            ).start()

            nxt = g + NBUF

            @pl.when(nxt < NSUB)
            def _():
                pltpu.make_async_copy(
                    table_hbm.at[idx_ref(g0 // 2 + (b + NBUF) // 2, b % 2)],
                    rin[b],
                    gsem[b],
                ).start()

    for b in range(NBUF):
        pltpu.make_async_copy(
            rout[b], out_hbm.at[NSUB - NBUF + b], ssem[b]
        ).wait()


_scratch = (
    [
        pltpu.VMEM((IDXROWS, 2 * SUB), jnp.int32),
        pltpu.VMEM((IDXROWS, 2 * SUB), jnp.int32),
        pltpu.VMEM((SEQ, 2 * DIM), jnp.float32),
    ]
    + [pltpu.VMEM((SUB, 2 * DIM), jnp.float32) for _ in range(NBUF)]
    + [pltpu.VMEM((SUB // 2, 2 * DIM), jnp.float32) for _ in range(NBUF)]
    + [pltpu.SemaphoreType.DMA for _ in range(2 * NBUF)]
)

_kern = pl.kernel(
    _body,
    out_type=jax.ShapeDtypeStruct((NW * NSUB, SUB // 2, 2 * DIM), jnp.float32),
    mesh=plsc.VectorSubcoreMesh(core_axis_name="c", subcore_axis_name="s"),
    scratch_types=_scratch,
    compiler_params=pltpu.CompilerParams(use_tc_tiling_on_sc=True),
    name="token_pos_embed_sc",
)


@jax.jit
def kernel(inputs, token_table, pos_table):
    b, l = inputs.shape
    _, d = token_table.shape
    ridx = inputs.astype(jnp.int32).reshape(NW, IDXROWS, 2 * SUB)
    pos2 = jnp.concatenate([pos_table, pos_table], axis=0).reshape(
        SEQ, 2 * DIM
    )
    out = _kern(ridx, token_table.reshape(-1, 2 * DIM), pos2)
    return out.reshape(b, l, d)
